# baseline (speedup 1.0000x reference)
import functools

import jax
import jax.numpy as jnp
from jax import lax
from jax.experimental import pallas as pl
from jax.experimental.pallas import tpu as pltpu
from jax.experimental.pallas import tpu_sc as plsc

L = 16


@functools.lru_cache(maxsize=None)
def _build_permute(rows: int, hw: int, chunk_rows: int):
    info = plsc.get_sparse_core_info()
    nc, ns = info.num_cores, info.num_subcores
    nw = nc * ns
    assert rows % (nw * chunk_rows) == 0
    rpw = rows // nw
    nchunk = rpw // chunk_rows
    nk = hw // L

    mesh = plsc.VectorSubcoreMesh(core_axis_name="c", subcore_axis_name="s")

    @functools.partial(
        pl.kernel,
        mesh=mesh,
        out_type=jax.ShapeDtypeStruct((rows * hw,), jnp.float32),
        scratch_types=[
            pltpu.VMEM((hw,), jnp.int32),
            pltpu.VMEM((chunk_rows * hw,), jnp.float32),
            pltpu.VMEM((chunk_rows * hw,), jnp.float32),
        ],
        compiler_params=pltpu.CompilerParams(needs_layout_passes=False),
    )
    def permute(x_hbm, idx_hbm, out_hbm, idx_v, in_v, out_v):
        wid = lax.axis_index("s") * nc + lax.axis_index("c")
        base = wid * rpw
        pltpu.sync_copy(idx_hbm, idx_v)

        def chunk_body(ci, carry):
            e0 = (base + ci * chunk_rows) * hw
            pltpu.sync_copy(x_hbm.at[pl.ds(e0, chunk_rows * hw)], in_v)

            def kbody(k, c2):
                idx = idx_v[pl.ds(k * L, L)]

                def rbody(r, c3):
                    vals = plsc.load_gather(in_v, [idx + r * hw])
                    out_v[pl.ds(r * hw + k * L, L)] = vals
                    return c3

                return lax.fori_loop(0, chunk_rows, rbody, c2, unroll=8)

            lax.fori_loop(0, nk, kbody, 0)
            pltpu.sync_copy(out_v, out_hbm.at[pl.ds(e0, chunk_rows * hw)])
            return carry

        lax.fori_loop(0, nchunk, chunk_body, 0)

    return permute


def kernel(x, rd_index):
    b, c, hw = x.shape
    rows = b * c
    permute = _build_permute(rows, hw, 24)
    out_flat = permute(x.reshape(rows * hw), rd_index)
    return out_flat.reshape(b, c, hw)


# speedup vs baseline: 1.4326x; 1.4326x over previous
import functools

import jax
import jax.numpy as jnp
from jax import lax
from jax.experimental import pallas as pl
from jax.experimental.pallas import tpu as pltpu
from jax.experimental.pallas import tpu_sc as plsc

L = 16
KG = 8


@functools.lru_cache(maxsize=None)
def _build_permute(rows: int, hw: int, chunk_rows: int):
    info = plsc.get_sparse_core_info()
    nc, ns = info.num_cores, info.num_subcores
    nw = nc * ns
    assert rows % (nw * chunk_rows) == 0
    rpw = rows // nw
    nchunk = rpw // chunk_rows
    nk = hw // L

    mesh = plsc.VectorSubcoreMesh(core_axis_name="c", subcore_axis_name="s")

    @functools.partial(
        pl.kernel,
        mesh=mesh,
        out_type=jax.ShapeDtypeStruct((rows * hw,), jnp.float32),
        scratch_types=[
            pltpu.VMEM((hw,), jnp.int32),
            pltpu.VMEM((chunk_rows * hw,), jnp.float32),
            pltpu.VMEM((chunk_rows * hw,), jnp.float32),
        ],
        compiler_params=pltpu.CompilerParams(needs_layout_passes=False),
    )
    def permute(x_hbm, idx_hbm, out_hbm, idx_v, in_v, out_v):
        wid = lax.axis_index("s") * nc + lax.axis_index("c")
        base = wid * rpw
        pltpu.sync_copy(idx_hbm, idx_v)

        def chunk_body(ci, carry):
            e0 = (base + ci * chunk_rows) * hw
            pltpu.sync_copy(x_hbm.at[pl.ds(e0, chunk_rows * hw)], in_v)

            def kbody(kb, c2):
                k0 = kb * KG
                idxs = [idx_v[pl.ds((k0 + j) * L, L)] for j in range(KG)]

                @plsc.parallel_loop(0, chunk_rows, unroll=2)
                def rbody(r):
                    row = in_v.at[pl.ds(r * hw, hw)]
                    for j in range(KG):
                        vals = plsc.load_gather(row, [idxs[j]])
                        out_v[pl.ds(r * hw + (k0 + j) * L, L)] = vals

                return c2

            lax.fori_loop(0, nk // KG, kbody, 0)
            pltpu.sync_copy(out_v, out_hbm.at[pl.ds(e0, chunk_rows * hw)])
            return carry

        lax.fori_loop(0, nchunk, chunk_body, 0)

    return permute


def kernel(x, rd_index):
    b, c, hw = x.shape
    rows = b * c
    permute = _build_permute(rows, hw, 24)
    out_flat = permute(x.reshape(rows * hw), rd_index)
    return out_flat.reshape(b, c, hw)


# speedup vs baseline: 2.7430x; 1.9147x over previous
import functools

import jax
import jax.numpy as jnp
from jax import lax
from jax.experimental import pallas as pl
from jax.experimental.pallas import tpu as pltpu
from jax.experimental.pallas import tpu_sc as plsc

L = 16
KG = 8
SR = 8


@functools.lru_cache(maxsize=None)
def _build_permute(stripes: int, hw: int, chunk_stripes: int):
    info = plsc.get_sparse_core_info()
    nc, ns = info.num_cores, info.num_subcores
    nw = nc * ns
    assert stripes % (nw * chunk_stripes) == 0
    spw = stripes // nw
    nchunk = spw // chunk_stripes
    nk = hw // L
    rows = chunk_stripes * SR

    mesh = plsc.VectorSubcoreMesh(core_axis_name="c", subcore_axis_name="s")

    @functools.partial(
        pl.kernel,
        mesh=mesh,
        out_type=jax.ShapeDtypeStruct((stripes, SR, hw), jnp.float32),
        scratch_types=[
            pltpu.VMEM((hw,), jnp.int32),
            pltpu.VMEM((chunk_stripes, SR, hw), jnp.float32),
            pltpu.VMEM((chunk_stripes, SR, hw), jnp.float32),
        ],
        compiler_params=pltpu.CompilerParams(
            needs_layout_passes=False, use_tc_tiling_on_sc=True
        ),
    )
    def permute(x_hbm, idx_hbm, out_hbm, idx_v, in_v, out_v):
        wid = lax.axis_index("s") * nc + lax.axis_index("c")
        base = wid * spw
        pltpu.sync_copy(idx_hbm, idx_v)

        def chunk_body(ci, carry):
            s0 = base + ci * chunk_stripes
            pltpu.sync_copy(x_hbm.at[pl.ds(s0, chunk_stripes)], in_v)

            def kbody(kb, c2):
                k0 = kb * KG
                idxs = [idx_v[pl.ds((k0 + j) * L, L)] for j in range(KG)]

                @plsc.parallel_loop(0, rows, unroll=2)
                def rbody(r):
                    s = r // SR
                    r8 = r % SR
                    sv = jnp.full((L,), s, jnp.int32)
                    rv = jnp.full((L,), r8, jnp.int32)
                    for j in range(KG):
                        vals = plsc.load_gather(in_v, [sv, rv, idxs[j]])
                        out_v[s, r8, pl.ds((k0 + j) * L, L)] = vals

                return c2

            lax.fori_loop(0, nk // KG, kbody, 0)
            pltpu.sync_copy(out_v, out_hbm.at[pl.ds(s0, chunk_stripes)])
            return carry

        lax.fori_loop(0, nchunk, chunk_body, 0)

    return permute


def kernel(x, rd_index):
    b, c, hw = x.shape
    stripes = b * c // SR
    permute = _build_permute(stripes, hw, 3)
    out = permute(x.reshape(stripes, SR, hw), rd_index)
    return out.reshape(b, c, hw)


# speedup vs baseline: 3.5377x; 1.2897x over previous
import functools

import jax
import jax.numpy as jnp
from jax import lax
from jax.experimental import pallas as pl
from jax.experimental.pallas import tpu as pltpu
from jax.experimental.pallas import tpu_sc as plsc

L = 16
KG = 8
SR = 8


@functools.lru_cache(maxsize=None)
def _build_permute(stripes: int, hw: int, chunk_stripes: int):
    info = plsc.get_sparse_core_info()
    nc, ns = info.num_cores, info.num_subcores
    nw = nc * ns
    assert stripes % (nw * chunk_stripes) == 0
    spw = stripes // nw
    nchunk = spw // chunk_stripes
    nk = hw // L
    rows = chunk_stripes * SR

    mesh = plsc.VectorSubcoreMesh(core_axis_name="c", subcore_axis_name="s")

    @functools.partial(
        pl.kernel,
        mesh=mesh,
        out_type=jax.ShapeDtypeStruct((stripes, SR, hw), jnp.float32),
        scratch_types=[
            pltpu.VMEM((hw,), jnp.int32),
            pltpu.VMEM((2, chunk_stripes, SR, hw), jnp.float32),
            pltpu.VMEM((2, chunk_stripes, SR, hw), jnp.float32),
            pltpu.SemaphoreType.DMA,
            pltpu.SemaphoreType.DMA,
            pltpu.SemaphoreType.DMA,
            pltpu.SemaphoreType.DMA,
        ],
        compiler_params=pltpu.CompilerParams(
            needs_layout_passes=False, use_tc_tiling_on_sc=True
        ),
    )
    def permute(x_hbm, idx_hbm, out_hbm, idx_v, in_v, out_v,
                sin0, sin1, sout0, sout1):
        wid = lax.axis_index("s") * nc + lax.axis_index("c")
        base = wid * spw
        sins = (sin0, sin1)
        souts = (sout0, sout1)
        pltpu.sync_copy(idx_hbm, idx_v)

        def in_copy(ci, b):
            s0 = base + ci * chunk_stripes
            return pltpu.make_async_copy(
                x_hbm.at[pl.ds(s0, chunk_stripes)], in_v.at[b], sins[b]
            )

        def out_copy(ci, b):
            s0 = base + ci * chunk_stripes
            return pltpu.make_async_copy(
                out_v.at[b], out_hbm.at[pl.ds(s0, chunk_stripes)], souts[b]
            )

        def compute(b):
            def kbody(kb, c2):
                k0 = kb * KG
                idxs = [idx_v[pl.ds((k0 + j) * L, L)] for j in range(KG)]

                @plsc.parallel_loop(0, rows, unroll=2)
                def rbody(r):
                    s = r // SR
                    r8 = r % SR
                    sv = jnp.full((L,), s, jnp.int32)
                    rv = jnp.full((L,), r8, jnp.int32)
                    for j in range(KG):
                        vals = plsc.load_gather(in_v.at[b], [sv, rv, idxs[j]])
                        out_v[b, s, r8, pl.ds((k0 + j) * L, L)] = vals

                return c2

            lax.fori_loop(0, nk // KG, kbody, 0)

        in_copy(0, 0).start()

        def pipe_body(i2, carry):
            for ph in range(2):
                ci = i2 * 2 + ph
                in_copy(ci, ph).wait()

                @pl.when(ci + 1 < nchunk)
                def _():
                    in_copy(ci + 1, 1 - ph).start()

                @pl.when(ci >= 2)
                def _():
                    out_copy(ci - 2, ph).wait()

                compute(ph)
                out_copy(ci, ph).start()
            return carry

        lax.fori_loop(0, nchunk // 2, pipe_body, 0)
        out_copy(nchunk - 2, 0).wait()
        out_copy(nchunk - 1, 1).wait()

    return permute


def kernel(x, rd_index):
    b, c, hw = x.shape
    stripes = b * c // SR
    permute = _build_permute(stripes, hw, 3)
    out = permute(x.reshape(stripes, SR, hw), rd_index)
    return out.reshape(b, c, hw)
